# baseline (speedup 1.0000x reference)
import functools

import jax
import jax.numpy as jnp
from jax import lax
from jax.experimental import pallas as pl
from jax.experimental.pallas import tpu as pltpu
from jax.experimental.pallas import tpu_sc as plsc

_VOCAB = 100000
_D = 1024
_MAX_SEQ = 2048
_B = 4
_S = 2048

_NC = 2
_NS = 16
_NW = _NC * _NS
_ROWS = _B * _S
_RPW = _ROWS // _NW
_C = 32
_NCH = _RPW // _C
_LANES_PER_ROW = _D // 16
_POS_SCALE = 1.0 / (_D ** 0.5)

_mesh = plsc.VectorSubcoreMesh(core_axis_name="c", subcore_axis_name="s")


@functools.partial(
    pl.kernel,
    out_type=jax.ShapeDtypeStruct((_ROWS, _D), jnp.float32),
    mesh=_mesh,
    scratch_types=[
        pltpu.VMEM((_RPW,), jnp.int32),
        pltpu.VMEM((_C, _D), jnp.float32),
        pltpu.VMEM((_C, _D), jnp.float32),
        pltpu.SemaphoreType.DMA,
    ],
)
def _embed(x_hbm, wt_hbm, pt_hbm, out_hbm, idx_v, rows_v, pos_v, sem):
    wid = lax.axis_index("s") * _NC + lax.axis_index("c")
    base = wid * _RPW
    s_base = base % _S

    pltpu.sync_copy(x_hbm.at[pl.ds(base, _RPW)], idx_v)

    for c in range(_NCH):
        off = c * _C
        gat = pltpu.async_copy(
            wt_hbm.at[idx_v.at[pl.ds(off, _C)]], rows_v, sem)
        pltpu.sync_copy(pt_hbm.at[pl.ds(s_base + off, _C)], pos_v)
        gat.wait()

        def row_body(i, carry):
            for j in range(_LANES_PER_ROW):
                sl = pl.ds(j * 16, 16)
                rows_v[i, sl] = rows_v[i, sl] + pos_v[i, sl] * _POS_SCALE
            return carry

        lax.fori_loop(0, _C, row_body, 0)

        pltpu.sync_copy(rows_v, out_hbm.at[pl.ds(base + off, _C)])


def kernel(x, word_table, pos_table):
    flat_idx = x.reshape(_ROWS).astype(jnp.int32)
    out = _embed(flat_idx, word_table, pos_table)
    return out.reshape(_B, _S, _D)


# speedup vs baseline: 1.0683x; 1.0683x over previous
import functools

import jax
import jax.numpy as jnp
from jax import lax
from jax.experimental import pallas as pl
from jax.experimental.pallas import tpu as pltpu
from jax.experimental.pallas import tpu_sc as plsc

_D = 1024
_B = 4
_S = 2048

_NC = 2
_NS = 16
_NW = _NC * _NS
_ROWS = _B * _S
_RPW = _ROWS // _NW
_C = 16
_NCH = _RPW // _C
_LPR = _D // 16
_POS_SCALE = 1.0 / (_D ** 0.5)

_mesh = plsc.VectorSubcoreMesh(core_axis_name="c", subcore_axis_name="s")


@functools.partial(
    pl.kernel,
    out_type=jax.ShapeDtypeStruct((_ROWS, _D), jnp.float32),
    mesh=_mesh,
    scratch_types=[
        pltpu.VMEM((_RPW,), jnp.int32),
        pltpu.VMEM((2, _C, _D), jnp.float32),
        pltpu.VMEM((2, _C, _D), jnp.float32),
        pltpu.SemaphoreType.DMA,
        pltpu.SemaphoreType.DMA,
    ],
)
def _embed(x_hbm, wt_hbm, pt_hbm, out_hbm, idx_v, rows_v, pos_v, gsem, psem):
    wid = lax.axis_index("s") * _NC + lax.axis_index("c")
    base = wid * _RPW
    s_base = base % _S

    pltpu.sync_copy(x_hbm.at[pl.ds(base, _RPW)], idx_v)

    def start(c, buf):
        off = c * _C
        g = pltpu.async_copy(
            wt_hbm.at[idx_v.at[pl.ds(off, _C)]], rows_v.at[buf], gsem)
        p = pltpu.async_copy(
            pt_hbm.at[pl.ds(s_base + off, _C)], pos_v.at[buf], psem)
        return g, p

    pend = start(0, 0)
    for c in range(_NCH):
        cur = c & 1
        g, p = pend
        g.wait()
        p.wait()
        if c + 1 < _NCH:
            pend = start(c + 1, 1 - cur)

        def row_body(i, carry):
            for j in range(_LPR):
                sl = pl.ds(j * 16, 16)
                rows_v[cur, i, sl] = (
                    rows_v[cur, i, sl] + pos_v[cur, i, sl] * _POS_SCALE)
            return carry

        lax.fori_loop(0, _C, row_body, 0)

        pltpu.sync_copy(rows_v.at[cur], out_hbm.at[pl.ds(base + c * _C, _C)])


def kernel(x, word_table, pos_table):
    flat_idx = x.reshape(_ROWS).astype(jnp.int32)
    out = _embed(flat_idx, word_table, pos_table)
    return out.reshape(_B, _S, _D)


# speedup vs baseline: 1.1207x; 1.0491x over previous
import functools

import jax
import jax.numpy as jnp
from jax import lax
from jax.experimental import pallas as pl
from jax.experimental.pallas import tpu as pltpu
from jax.experimental.pallas import tpu_sc as plsc

_D = 1024
_B = 4
_S = 2048

_NC = 2
_NS = 16
_NW = _NC * _NS
_ROWS = _B * _S
_SPW = _S // _NW
_CP = 8
_NCH = _SPW // _CP
_CR = _B * _CP
_POS_SCALE = 1.0 / (_D ** 0.5)

_mesh = plsc.VectorSubcoreMesh(core_axis_name="c", subcore_axis_name="s")


@functools.partial(
    pl.kernel,
    out_type=jax.ShapeDtypeStruct((_ROWS, _D), jnp.float32),
    mesh=_mesh,
    scratch_types=[
        pltpu.VMEM((_B * _SPW,), jnp.int32),
        pltpu.VMEM((3, _CR, _D), jnp.float32),
        pltpu.VMEM((2, _CP, _D), jnp.float32),
        pltpu.SemaphoreType.DMA,
        pltpu.SemaphoreType.DMA,
        pltpu.SemaphoreType.DMA,
    ],
)
def _embed(x_hbm, wt_hbm, pt_hbm, out_hbm, idx_v, rows_v, pos_v, gsem, psem,
           osem):
    wid = lax.axis_index("s") * _NC + lax.axis_index("c")
    s0 = wid * _SPW

    for b in range(_B):
        pltpu.sync_copy(x_hbm.at[pl.ds(b * _S + s0, _SPW)],
                        idx_v.at[pl.ds(b * _SPW, _SPW)])

    def start_in(c, bi, pb):
        waits = [pltpu.async_copy(
            pt_hbm.at[pl.ds(s0 + c * _CP, _CP)], pos_v.at[pb], psem)]
        for b in range(_B):
            waits.append(pltpu.async_copy(
                wt_hbm.at[idx_v.at[pl.ds(b * _SPW + c * _CP, _CP)]],
                rows_v.at[bi, pl.ds(b * _CP, _CP)], gsem))
        return waits

    def start_out(c, bi):
        return [pltpu.async_copy(
            rows_v.at[bi, pl.ds(b * _CP, _CP)],
            out_hbm.at[pl.ds(b * _S + s0 + c * _CP, _CP)], osem)
            for b in range(_B)]

    pend_in = start_in(0, 0, 0)
    pend_out = {}
    for c in range(_NCH):
        bi = c % 3
        pb = c & 1
        for w in pend_in:
            w.wait()
        if c + 1 < _NCH:
            if c >= 2:
                for w in pend_out.pop(c - 2):
                    w.wait()
            pend_in = start_in(c + 1, (c + 1) % 3, (c + 1) & 1)

        def i_body(i, carry):
            def jo_body(jo, carry2):
                for jj in range(16):
                    sl = pl.ds(jo * 256 + jj * 16, 16)
                    p = pos_v[pb, i, sl] * _POS_SCALE
                    for b in range(_B):
                        r = b * _CP + i
                        rows_v[bi, r, sl] = rows_v[bi, r, sl] + p
                return carry2
            return lax.fori_loop(0, 4, jo_body, carry)

        lax.fori_loop(0, _CP, i_body, 0)

        pend_out[c] = start_out(c, bi)

    for c in sorted(pend_out):
        for w in pend_out[c]:
            w.wait()


def kernel(x, word_table, pos_table):
    flat_idx = x.reshape(_ROWS).astype(jnp.int32)
    out = _embed(flat_idx, word_table, pos_table)
    return out.reshape(_B, _S, _D)


# speedup vs baseline: 1.6980x; 1.5151x over previous
import functools

import jax
import jax.numpy as jnp
from jax import lax
from jax.experimental import pallas as pl
from jax.experimental.pallas import tpu as pltpu
from jax.experimental.pallas import tpu_sc as plsc

_D = 1024
_B = 4
_S = 2048

_NC = 2
_NS = 16
_NW = _NC * _NS
_ROWS = _B * _S
_SPW = _S // _NW
_CP = 8
_NCH = _SPW // _CP
_CR = _B * _CP
_POS_SCALE = 1.0 / (_D ** 0.5)

_mesh = plsc.VectorSubcoreMesh(core_axis_name="c", subcore_axis_name="s")


@functools.partial(
    pl.kernel,
    out_type=jax.ShapeDtypeStruct((_ROWS, _D), jnp.float32),
    mesh=_mesh,
    scratch_types=[
        pltpu.VMEM((_B * _SPW,), jnp.int32),
        pltpu.VMEM((3, _CR, _D), jnp.float32),
        pltpu.VMEM((2, _CP, _D), jnp.float32),
        pltpu.SemaphoreType.DMA,
        pltpu.SemaphoreType.DMA,
        pltpu.SemaphoreType.DMA,
    ],
)
def _embed(x_hbm, wt_hbm, pt_hbm, out_hbm, idx_v, rows_v, pos_v, gsem, psem,
           osem):
    wid = lax.axis_index("s") * _NC + lax.axis_index("c")
    s0 = wid * _SPW

    pltpu.sync_copy(x_hbm.at[pl.ds(wid * _B * _SPW, _B * _SPW)], idx_v)

    def start_in(c, bi, pb):
        return [
            pltpu.async_copy(
                pt_hbm.at[pl.ds(s0 + c * _CP, _CP)], pos_v.at[pb], psem),
            pltpu.async_copy(
                wt_hbm.at[idx_v.at[pl.ds(c * _CR, _CR)]], rows_v.at[bi], gsem),
        ]

    def start_out(c, bi):
        return [pltpu.async_copy(
            rows_v.at[bi, pl.ds(b * _CP, _CP)],
            out_hbm.at[pl.ds(b * _S + s0 + c * _CP, _CP)], osem)
            for b in range(_B)]

    pend_in = start_in(0, 0, 0)
    pend_out = {}
    for c in range(_NCH):
        bi = c % 3
        pb = c & 1
        for w in pend_in:
            w.wait()
        if c + 1 < _NCH:
            if c >= 2:
                for w in pend_out.pop(c - 2):
                    w.wait()
            pend_in = start_in(c + 1, (c + 1) % 3, (c + 1) & 1)

        def i_body(i, carry):
            for j in range(_D // 16):
                sl = pl.ds(j * 16, 16)
                p = pos_v[pb, i, sl] * _POS_SCALE
                for b in range(_B):
                    r = b * _CP + i
                    rows_v[bi, r, sl] = rows_v[bi, r, sl] + p
            return carry

        lax.fori_loop(0, _CP, i_body, 0)

        pend_out[c] = start_out(c, bi)

    for c in sorted(pend_out):
        for w in pend_out[c]:
            w.wait()


def kernel(x, word_table, pos_table):
    flat_idx = (x.astype(jnp.int32)
                .reshape(_B, _NW, _NCH, _CP)
                .transpose(1, 2, 0, 3)
                .reshape(_ROWS))
    out = _embed(flat_idx, word_table, pos_table)
    return out.reshape(_B, _S, _D)
